# baseline (speedup 1.0000x reference)
import functools
import math

import jax
import jax.numpy as jnp
from jax import lax
from jax.experimental import pallas as pl
from jax.experimental.pallas import tpu as pltpu
from jax.experimental.pallas import tpu_sc as plsc

D = 1024
B = 4
T = 8192
K = 64

NC = 2
NS = 16
NW = NC * NS
ROWS = B * K
R_PER_W = ROWS // NW


T_SC = 4096
T_TC = T - T_SC


def _mean_body(p_ref, out_ref):
    out_ref[...] = jnp.mean(p_ref[...], axis=-1)


def _p_mean_tc(p_all, bt=256):
    grid = (T_TC // bt,)
    return pl.pallas_call(
        _mean_body,
        grid=grid,
        in_specs=[pl.BlockSpec((B, bt, D), lambda i: (0, i, 0))],
        out_specs=pl.BlockSpec((B, bt), lambda i: (0, i)),
        out_shape=jax.ShapeDtypeStruct((B, T_TC), jnp.float32),
    )(p_all)


RPW = T_SC // 8
CH = 32
NCHK = RPW // CH


UNROLL = 16


def _sc_mean(p_1d):
    mesh = plsc.VectorSubcoreMesh(core_axis_name="c", subcore_axis_name="s")

    @functools.partial(
        pl.kernel,
        mesh=mesh,
        compiler_params=pltpu.CompilerParams(needs_layout_passes=False),
        out_type=jax.ShapeDtypeStruct((B * T_SC,), jnp.float32),
        scratch_types=[
            pltpu.VMEM((CH, D), jnp.float32),
            pltpu.VMEM((CH, D), jnp.float32),
            pltpu.VMEM((RPW,), jnp.float32),
            pltpu.SemaphoreType.DMA,
            pltpu.SemaphoreType.DMA,
        ],
    )
    def mean_k(p_hbm, out_hbm, buf0, buf1, mbuf, sem0, sem1):
        w = lax.axis_index("s") * NC + lax.axis_index("c")
        b = w // 8
        j = w % 8
        row0 = b * T + T_TC + j * RPW
        obase = b * T_SC + j * RPW
        lane16 = lax.broadcasted_iota(jnp.int32, (16,), 0)
        lane0 = lane16 * 0
        inv_d = jnp.float32(1.0 / D)
        bufs = (buf0, buf1)
        sems = (sem0, sem1)

        dmas = [None, None]
        dmas[0] = pltpu.async_copy(p_hbm.at[pl.ds(row0, CH)], buf0, sem0)
        for g in range(NCHK):
            s = g % 2
            if g + 1 < NCHK:
                ns = (g + 1) % 2
                dmas[ns] = pltpu.async_copy(
                    p_hbm.at[pl.ds(row0 + (g + 1) * CH, CH)],
                    bufs[ns], sems[ns])
            dmas[s].wait()
            bufg = bufs[s]
            for grp in range(CH // 16):
                ridx = lane16 + grp * 16

                def col_body(i0, acc, bufg=bufg, ridx=ridx):
                    for di in range(UNROLL):
                        acc = acc + plsc.load_gather(
                            bufg, [ridx, lane0 + (i0 * UNROLL + di)])
                    return acc

                acc = lax.fori_loop(0, D // UNROLL, col_body,
                                    jnp.zeros((16,), jnp.float32))
                mbuf[pl.ds(g * CH + grp * 16, 16)] = acc * inv_d
        pltpu.sync_copy(mbuf, out_hbm.at[pl.ds(obase, RPW)])

    return mean_k(p_1d)


def _topk_body(ptc_ref, psc_ref, out_ref):
    p_cur = jnp.concatenate([ptc_ref[...], psc_ref[...]], axis=1)
    iota = lax.broadcasted_iota(jnp.int32, (B, T), 1)
    cols = []
    for _ in range(K):
        vmax = jnp.max(p_cur, axis=1, keepdims=True)
        eq = p_cur == vmax
        idx = jnp.min(jnp.where(eq, iota, T), axis=1, keepdims=True)
        cols.append(idx)
        p_cur = jnp.where(iota == idx, jnp.float32(-1.0), p_cur)
    idxs = jnp.concatenate(cols, axis=1)
    offs = lax.broadcasted_iota(jnp.int32, (B, K), 0) * T
    out_ref[...] = idxs + offs


def _topk_flat_idx(p_tc, p_sc):
    return pl.pallas_call(
        _topk_body,
        out_shape=jax.ShapeDtypeStruct((B, K), jnp.int32),
    )(p_tc, p_sc)


def _sc_gather(table, flat_idx):
    mesh = plsc.VectorSubcoreMesh(core_axis_name="c", subcore_axis_name="s")

    @functools.partial(
        pl.kernel,
        mesh=mesh,
        out_type=jax.ShapeDtypeStruct((ROWS, D), jnp.float32),
        scratch_types=[
            pltpu.VMEM((R_PER_W,), jnp.int32),
            pltpu.VMEM((R_PER_W, D), jnp.float32),
            pltpu.SemaphoreType.DMA,
        ],
    )
    def gather_k(table_hbm, idx_hbm, out_hbm, idx_v, rows_v, sem):
        wid = lax.axis_index("s") * NC + lax.axis_index("c")
        base = wid * R_PER_W
        pltpu.sync_copy(idx_hbm.at[pl.ds(base, R_PER_W)], idx_v)
        pltpu.async_copy(table_hbm.at[idx_v], rows_v, sem).wait()
        pltpu.sync_copy(rows_v, out_hbm.at[pl.ds(base, R_PER_W)])

    return gather_k(table, flat_idx)


def _dense_body(ht_ref, hm_ref, wq_ref, bq_ref, wk_ref, bk_ref,
                wv_ref, bv_ref, wg_ref, bg_ref, out_ref):
    hm = hm_ref[...]
    cdims = (((1,), (1,)), ((), ()))
    q = lax.dot_general(hm, wq_ref[...], cdims,
                        preferred_element_type=jnp.float32) + bq_ref[...]
    qk = jnp.dot(q, wk_ref[...], preferred_element_type=jnp.float32)
    qbk = jnp.sum(q * bk_ref[...], axis=1, keepdims=True)
    ht3 = ht_ref[...].reshape(B, K, D)

    scale = 1.0 / math.sqrt(D)
    scores = (jnp.sum(ht3 * qk[:, None, :], axis=-1) + qbk) * scale
    m = jnp.max(scores, axis=-1, keepdims=True)
    e = jnp.exp(scores - m)
    attn = e / jnp.sum(e, axis=-1, keepdims=True)
    mix = jnp.sum(ht3 * attn[:, :, None], axis=1)
    cache = lax.dot_general(mix, wv_ref[...], cdims,
                            preferred_element_type=jnp.float32) + bv_ref[...]
    g_lin = jnp.sum(hm * wg_ref[...], axis=1, keepdims=True) + bg_ref[...]
    g = 1.0 / (1.0 + jnp.exp(-g_lin))
    out_ref[...] = hm + g * cache


def _dense(h_topk, h_mean, W_q, b_q, W_k, b_k, W_v, b_v, W_g, b_g):
    return pl.pallas_call(
        _dense_body,
        out_shape=jax.ShapeDtypeStruct((B, D), jnp.float32),
    )(h_topk, h_mean, W_q, b_q.reshape(1, D), W_k, b_k.reshape(1, D),
      W_v, b_v.reshape(1, D), W_g, b_g.reshape(1, 1))


def kernel(h_mean, h_all, p_all, W_k, b_k, W_v, b_v, W_q, b_q, W_g, b_g):
    p_tc = _p_mean_tc(p_all)
    p_sc = _sc_mean(p_all.reshape(B * T, D)).reshape(B, T_SC)
    flat_idx = _topk_flat_idx(p_tc, p_sc).reshape(ROWS)
    h_topk = _sc_gather(h_all.reshape(B * T, D), flat_idx)
    return _dense(h_topk, h_mean, W_q, b_q, W_k, b_k, W_v, b_v, W_g, b_g)


# speedup vs baseline: 2.7443x; 2.7443x over previous
import functools
import math

import jax
import jax.numpy as jnp
from jax import lax
from jax.experimental import pallas as pl
from jax.experimental.pallas import tpu as pltpu
from jax.experimental.pallas import tpu_sc as plsc

D = 1024
B = 4
T = 8192
K = 64

NC = 2
NS = 16
NW = NC * NS
ROWS = B * K
R_PER_W = ROWS // NW


T_SC = 4096
T_TC = T - T_SC


def _mean_body(p_ref, out_ref):
    out_ref[...] = jnp.mean(p_ref[...], axis=-1)


def _p_mean_tc(p_all, bt=256):
    grid = (T_TC // bt,)
    return pl.pallas_call(
        _mean_body,
        grid=grid,
        in_specs=[pl.BlockSpec((B, bt, D), lambda i: (0, i, 0))],
        out_specs=pl.BlockSpec((B, bt), lambda i: (0, i)),
        out_shape=jax.ShapeDtypeStruct((B, T_TC), jnp.float32),
    )(p_all)


RPW = T_SC // 8
CH = 32
NCHK = RPW // CH


UNROLL = 16


def _sc_mean(p_1d):
    mesh = plsc.VectorSubcoreMesh(core_axis_name="c", subcore_axis_name="s")

    @functools.partial(
        pl.kernel,
        mesh=mesh,
        compiler_params=pltpu.CompilerParams(needs_layout_passes=False),
        out_type=jax.ShapeDtypeStruct((B * T_SC,), jnp.float32),
        scratch_types=[
            pltpu.VMEM((CH, D), jnp.float32),
            pltpu.VMEM((CH, D), jnp.float32),
            pltpu.VMEM((RPW,), jnp.float32),
            pltpu.VMEM((16, 16), jnp.float32),
            pltpu.SemaphoreType.DMA,
            pltpu.SemaphoreType.DMA,
        ],
    )
    def mean_k(p_hbm, out_hbm, buf0, buf1, mbuf, abuf, sem0, sem1):
        w = lax.axis_index("s") * NC + lax.axis_index("c")
        b = w // 8
        j = w % 8
        row0 = b * T + T_TC + j * RPW
        obase = b * T_SC + j * RPW
        lane16 = lax.broadcasted_iota(jnp.int32, (16,), 0)
        lane0 = lane16 * 0
        inv_d = jnp.float32(1.0 / D)
        bufs = (buf0, buf1)
        sems = (sem0, sem1)

        dmas = [None, None]
        dmas[0] = pltpu.async_copy(p_hbm.at[pl.ds(row0, CH)], buf0, sem0)
        for g in range(NCHK):
            s = g % 2
            if g + 1 < NCHK:
                ns = (g + 1) % 2
                dmas[ns] = pltpu.async_copy(
                    p_hbm.at[pl.ds(row0 + (g + 1) * CH, CH)],
                    bufs[ns], sems[ns])
            dmas[s].wait()
            bufg = bufs[s]
            for grp in range(CH // 16):
                base = grp * 16

                def row_body(r, c, bufg=bufg, base=base):
                    row = base + r
                    a0 = bufg[row, pl.ds(0, 16)]
                    a1 = bufg[row, pl.ds(16, 16)]
                    a2 = bufg[row, pl.ds(32, 16)]
                    a3 = bufg[row, pl.ds(48, 16)]
                    for i in range(4, D // 16, 4):
                        a0 = a0 + bufg[row, pl.ds(i * 16, 16)]
                        a1 = a1 + bufg[row, pl.ds(i * 16 + 16, 16)]
                        a2 = a2 + bufg[row, pl.ds(i * 16 + 32, 16)]
                        a3 = a3 + bufg[row, pl.ds(i * 16 + 48, 16)]
                    abuf[r, :] = (a0 + a1) + (a2 + a3)
                    return c

                lax.fori_loop(0, 16, row_body, 0)
                t0 = plsc.load_gather(abuf, [lane16, lane0])
                t1 = plsc.load_gather(abuf, [lane16, lane0 + 1])
                t2 = plsc.load_gather(abuf, [lane16, lane0 + 2])
                t3 = plsc.load_gather(abuf, [lane16, lane0 + 3])
                for c in range(4, 16, 4):
                    t0 = t0 + plsc.load_gather(abuf, [lane16, lane0 + c])
                    t1 = t1 + plsc.load_gather(abuf, [lane16, lane0 + c + 1])
                    t2 = t2 + plsc.load_gather(abuf, [lane16, lane0 + c + 2])
                    t3 = t3 + plsc.load_gather(abuf, [lane16, lane0 + c + 3])
                tot = (t0 + t1) + (t2 + t3)
                mbuf[pl.ds(g * CH + base, 16)] = tot * inv_d
        pltpu.sync_copy(mbuf, out_hbm.at[pl.ds(obase, RPW)])

    return mean_k(p_1d)


def _topk_body(ptc_ref, psc_ref, out_ref):
    p_cur = jnp.concatenate([ptc_ref[...], psc_ref[...]], axis=1)
    iota = lax.broadcasted_iota(jnp.int32, (B, T), 1)
    cols = []
    for _ in range(K):
        vmax = jnp.max(p_cur, axis=1, keepdims=True)
        eq = p_cur == vmax
        idx = jnp.min(jnp.where(eq, iota, T), axis=1, keepdims=True)
        cols.append(idx)
        p_cur = jnp.where(iota == idx, jnp.float32(-1.0), p_cur)
    idxs = jnp.concatenate(cols, axis=1)
    offs = lax.broadcasted_iota(jnp.int32, (B, K), 0) * T
    out_ref[...] = idxs + offs


def _topk_flat_idx(p_tc, p_sc):
    return pl.pallas_call(
        _topk_body,
        out_shape=jax.ShapeDtypeStruct((B, K), jnp.int32),
    )(p_tc, p_sc)


def _sc_gather(table, flat_idx):
    mesh = plsc.VectorSubcoreMesh(core_axis_name="c", subcore_axis_name="s")

    @functools.partial(
        pl.kernel,
        mesh=mesh,
        out_type=jax.ShapeDtypeStruct((ROWS, D), jnp.float32),
        scratch_types=[
            pltpu.VMEM((R_PER_W,), jnp.int32),
            pltpu.VMEM((R_PER_W, D), jnp.float32),
            pltpu.SemaphoreType.DMA,
        ],
    )
    def gather_k(table_hbm, idx_hbm, out_hbm, idx_v, rows_v, sem):
        wid = lax.axis_index("s") * NC + lax.axis_index("c")
        base = wid * R_PER_W
        pltpu.sync_copy(idx_hbm.at[pl.ds(base, R_PER_W)], idx_v)
        pltpu.async_copy(table_hbm.at[idx_v], rows_v, sem).wait()
        pltpu.sync_copy(rows_v, out_hbm.at[pl.ds(base, R_PER_W)])

    return gather_k(table, flat_idx)


def _dense_body(ht_ref, hm_ref, wq_ref, bq_ref, wk_ref, bk_ref,
                wv_ref, bv_ref, wg_ref, bg_ref, out_ref):
    hm = hm_ref[...]
    cdims = (((1,), (1,)), ((), ()))
    q = lax.dot_general(hm, wq_ref[...], cdims,
                        preferred_element_type=jnp.float32) + bq_ref[...]
    qk = jnp.dot(q, wk_ref[...], preferred_element_type=jnp.float32)
    qbk = jnp.sum(q * bk_ref[...], axis=1, keepdims=True)
    ht3 = ht_ref[...].reshape(B, K, D)

    scale = 1.0 / math.sqrt(D)
    scores = (jnp.sum(ht3 * qk[:, None, :], axis=-1) + qbk) * scale
    m = jnp.max(scores, axis=-1, keepdims=True)
    e = jnp.exp(scores - m)
    attn = e / jnp.sum(e, axis=-1, keepdims=True)
    mix = jnp.sum(ht3 * attn[:, :, None], axis=1)
    cache = lax.dot_general(mix, wv_ref[...], cdims,
                            preferred_element_type=jnp.float32) + bv_ref[...]
    g_lin = jnp.sum(hm * wg_ref[...], axis=1, keepdims=True) + bg_ref[...]
    g = 1.0 / (1.0 + jnp.exp(-g_lin))
    out_ref[...] = hm + g * cache


def _dense(h_topk, h_mean, W_q, b_q, W_k, b_k, W_v, b_v, W_g, b_g):
    return pl.pallas_call(
        _dense_body,
        out_shape=jax.ShapeDtypeStruct((B, D), jnp.float32),
    )(h_topk, h_mean, W_q, b_q.reshape(1, D), W_k, b_k.reshape(1, D),
      W_v, b_v.reshape(1, D), W_g, b_g.reshape(1, 1))


def kernel(h_mean, h_all, p_all, W_k, b_k, W_v, b_v, W_q, b_q, W_g, b_g):
    p_sc = _sc_mean(p_all.reshape(B * T, D)).reshape(B, T_SC)
    p_tc = _p_mean_tc(p_all)
    flat_idx = _topk_flat_idx(p_tc, p_sc).reshape(ROWS)
    h_topk = _sc_gather(h_all.reshape(B * T, D), flat_idx)
    return _dense(h_topk, h_mean, W_q, b_q, W_k, b_k, W_v, b_v, W_g, b_g)


# speedup vs baseline: 2.9922x; 1.0904x over previous
import functools
import math

import jax
import jax.numpy as jnp
from jax import lax
from jax.experimental import pallas as pl
from jax.experimental.pallas import tpu as pltpu
from jax.experimental.pallas import tpu_sc as plsc

D = 1024
B = 4
T = 8192
K = 64

NC = 2
NS = 16
NW = NC * NS
ROWS = B * K
R_PER_W = ROWS // NW


T_SC = 3072
T_TC = T - T_SC


def _mean_body(p_ref, out_ref):
    out_ref[...] = jnp.mean(p_ref[...], axis=-1)


def _p_mean_tc(p_all, bt=256):
    grid = (T_TC // bt,)
    return pl.pallas_call(
        _mean_body,
        grid=grid,
        in_specs=[pl.BlockSpec((B, bt, D), lambda i: (0, i, 0))],
        out_specs=pl.BlockSpec((B, bt), lambda i: (0, i)),
        out_shape=jax.ShapeDtypeStruct((B, T_TC), jnp.float32),
    )(p_all)


RPW = T_SC // 8
CH = 32
NCHK = RPW // CH


UNROLL = 16


def _sc_mean(p_1d):
    mesh = plsc.VectorSubcoreMesh(core_axis_name="c", subcore_axis_name="s")

    @functools.partial(
        pl.kernel,
        mesh=mesh,
        compiler_params=pltpu.CompilerParams(needs_layout_passes=False),
        out_type=jax.ShapeDtypeStruct((B, T_SC), jnp.float32),
        scratch_types=[
            pltpu.VMEM((CH, D), jnp.float32),
            pltpu.VMEM((CH, D), jnp.float32),
            pltpu.VMEM((RPW,), jnp.float32),
            pltpu.VMEM((16, 16), jnp.float32),
            pltpu.SemaphoreType.DMA,
            pltpu.SemaphoreType.DMA,
        ],
    )
    def mean_k(p_hbm, out_hbm, buf0, buf1, mbuf, abuf, sem0, sem1):
        w = lax.axis_index("s") * NC + lax.axis_index("c")
        b = w // 8
        j = w % 8
        row0 = b * T + T_TC + j * RPW
        obase = j * RPW
        lane16 = lax.broadcasted_iota(jnp.int32, (16,), 0)
        lane0 = lane16 * 0
        inv_d = jnp.float32(1.0 / D)
        bufs = (buf0, buf1)
        sems = (sem0, sem1)

        dmas = [None, None]
        dmas[0] = pltpu.async_copy(p_hbm.at[pl.ds(row0, CH)], buf0, sem0)
        for g in range(NCHK):
            s = g % 2
            if g + 1 < NCHK:
                ns = (g + 1) % 2
                dmas[ns] = pltpu.async_copy(
                    p_hbm.at[pl.ds(row0 + (g + 1) * CH, CH)],
                    bufs[ns], sems[ns])
            dmas[s].wait()
            bufg = bufs[s]
            for grp in range(CH // 16):
                base = grp * 16

                def row_body(r, c, bufg=bufg, base=base):
                    row = base + r
                    a0 = bufg[row, pl.ds(0, 16)]
                    a1 = bufg[row, pl.ds(16, 16)]
                    a2 = bufg[row, pl.ds(32, 16)]
                    a3 = bufg[row, pl.ds(48, 16)]
                    for i in range(4, D // 16, 4):
                        a0 = a0 + bufg[row, pl.ds(i * 16, 16)]
                        a1 = a1 + bufg[row, pl.ds(i * 16 + 16, 16)]
                        a2 = a2 + bufg[row, pl.ds(i * 16 + 32, 16)]
                        a3 = a3 + bufg[row, pl.ds(i * 16 + 48, 16)]
                    abuf[r, :] = (a0 + a1) + (a2 + a3)
                    return c

                lax.fori_loop(0, 16, row_body, 0)
                t0 = plsc.load_gather(abuf, [lane16, lane0])
                t1 = plsc.load_gather(abuf, [lane16, lane0 + 1])
                t2 = plsc.load_gather(abuf, [lane16, lane0 + 2])
                t3 = plsc.load_gather(abuf, [lane16, lane0 + 3])
                for c in range(4, 16, 4):
                    t0 = t0 + plsc.load_gather(abuf, [lane16, lane0 + c])
                    t1 = t1 + plsc.load_gather(abuf, [lane16, lane0 + c + 1])
                    t2 = t2 + plsc.load_gather(abuf, [lane16, lane0 + c + 2])
                    t3 = t3 + plsc.load_gather(abuf, [lane16, lane0 + c + 3])
                tot = (t0 + t1) + (t2 + t3)
                mbuf[pl.ds(g * CH + base, 16)] = tot * inv_d
        pltpu.sync_copy(mbuf, out_hbm.at[b, pl.ds(obase, RPW)])

    return mean_k(p_1d)


def _topk_body(ptc_ref, psc_ref, out_ref):
    p_cur = jnp.concatenate([ptc_ref[...], psc_ref[...]], axis=1)
    iota = lax.broadcasted_iota(jnp.int32, (B, T), 1)
    cols = []
    for _ in range(K):
        vmax = jnp.max(p_cur, axis=1, keepdims=True)
        eq = p_cur == vmax
        idx = jnp.min(jnp.where(eq, iota, T), axis=1, keepdims=True)
        cols.append(idx)
        p_cur = jnp.where(iota == idx, jnp.float32(-1.0), p_cur)
    idxs = jnp.concatenate(cols, axis=1)
    offs = lax.broadcasted_iota(jnp.int32, (B, K), 0) * T
    out_ref[...] = idxs + offs


def _topk_flat_idx(p_tc, p_sc):
    return pl.pallas_call(
        _topk_body,
        out_shape=jax.ShapeDtypeStruct((B, K), jnp.int32),
    )(p_tc, p_sc)


def _sc_gather(table, flat_idx):
    mesh = plsc.VectorSubcoreMesh(core_axis_name="c", subcore_axis_name="s")

    @functools.partial(
        pl.kernel,
        mesh=mesh,
        out_type=jax.ShapeDtypeStruct((ROWS, D), jnp.float32),
        scratch_types=[
            pltpu.VMEM((R_PER_W,), jnp.int32),
            pltpu.VMEM((R_PER_W, D), jnp.float32),
            pltpu.SemaphoreType.DMA,
        ],
    )
    def gather_k(table_hbm, idx_hbm, out_hbm, idx_v, rows_v, sem):
        wid = lax.axis_index("s") * NC + lax.axis_index("c")
        base = wid * R_PER_W
        bb = wid // 8
        seg = wid % 8
        pltpu.sync_copy(idx_hbm.at[bb, pl.ds(seg * R_PER_W, R_PER_W)], idx_v)
        pltpu.async_copy(table_hbm.at[idx_v], rows_v, sem).wait()
        pltpu.sync_copy(rows_v, out_hbm.at[pl.ds(base, R_PER_W)])

    return gather_k(table, flat_idx)


def _prologue_body(hm_ref, wq_ref, bq_ref, wk_ref, bk_ref, wg_ref, bg_ref,
                   qk_ref, aux_ref):
    hm = hm_ref[...]
    cdims = (((1,), (1,)), ((), ()))
    q = lax.dot_general(hm, wq_ref[...], cdims,
                        preferred_element_type=jnp.float32) + bq_ref[...]
    qk_ref[...] = jnp.dot(q, wk_ref[...], preferred_element_type=jnp.float32)
    qbk = jnp.sum(q * bk_ref[...], axis=1, keepdims=True)
    g_lin = jnp.sum(hm * wg_ref[...], axis=1, keepdims=True) + bg_ref[...]
    lane = lax.broadcasted_iota(jnp.int32, (B, 128), 1)
    aux_ref[...] = (jnp.where(lane == 0, qbk, 0.0)
                    + jnp.where(lane == 1, g_lin, 0.0))


def _prologue(h_mean, W_q, b_q, W_k, b_k, W_g, b_g):
    return pl.pallas_call(
        _prologue_body,
        out_shape=(jax.ShapeDtypeStruct((B, D), jnp.float32),
                   jax.ShapeDtypeStruct((B, 128), jnp.float32)),
    )(h_mean, W_q, b_q.reshape(1, D), W_k, b_k.reshape(1, D),
      W_g, b_g.reshape(1, 1))


def _dense_body(ht_ref, hm_ref, qk_ref, aux_ref, wv_ref, bv_ref, out_ref):
    hm = hm_ref[...]
    cdims = (((1,), (1,)), ((), ()))
    qk = qk_ref[...]
    qbk = aux_ref[:, 0:1]
    g_lin = aux_ref[:, 1:2]
    ht3 = ht_ref[...].reshape(B, K, D)

    scale = 1.0 / math.sqrt(D)
    scores = (jnp.sum(ht3 * qk[:, None, :], axis=-1) + qbk) * scale
    m = jnp.max(scores, axis=-1, keepdims=True)
    e = jnp.exp(scores - m)
    attn = e / jnp.sum(e, axis=-1, keepdims=True)
    mix = jnp.sum(ht3 * attn[:, :, None], axis=1)
    cache = lax.dot_general(mix, wv_ref[...], cdims,
                            preferred_element_type=jnp.float32) + bv_ref[...]
    g = 1.0 / (1.0 + jnp.exp(-g_lin))
    out_ref[...] = hm + g * cache


def _dense(h_topk, h_mean, qk, aux, W_v, b_v):
    return pl.pallas_call(
        _dense_body,
        out_shape=jax.ShapeDtypeStruct((B, D), jnp.float32),
    )(h_topk, h_mean, qk, aux, W_v, b_v.reshape(1, D))


def kernel(h_mean, h_all, p_all, W_k, b_k, W_v, b_v, W_q, b_q, W_g, b_g):
    p_sc = _sc_mean(p_all.reshape(B * T, D))
    p_tc = _p_mean_tc(p_all)
    qk, aux = _prologue(h_mean, W_q, b_q, W_k, b_k, W_g, b_g)
    idx2d = _topk_flat_idx(p_tc, p_sc)
    h_topk = _sc_gather(h_all.reshape(B * T, D), idx2d)
    return _dense(h_topk, h_mean, qk, aux, W_v, b_v)
